# baseline (speedup 1.0000x reference)
import jax
import jax.numpy as jnp
from jax import lax
from jax.experimental import pallas as pl
from jax.experimental.pallas import tpu as pltpu

N = 4096
D = 512
RB = 256
NB = N // RB


def _body(spref, m_ref, x_ref, lmT_ref,
          W1_ref, b1_ref, W2_ref, b2_ref, Wp_ref, bp_ref, out_ref,
          deg_s, row_s):
    i = pl.program_id(0)
    idx = spref[0]

    @pl.when(i < NB)
    def _deg_phase():
        mb = m_ref[...]
        ones8 = jnp.ones((N, 8), dtype=jnp.float32)
        deg_s[pl.ds(i * RB, RB), :] = jnp.dot(
            mb, ones8, preferred_element_type=jnp.float32)
        rel = idx - i * RB

        @pl.when((rel >= 0) & (rel < RB))
        def _extract_row():
            sel = (lax.broadcasted_iota(jnp.int32, (1, RB), 1)
                   == rel).astype(jnp.float32)
            rowc = jnp.dot(sel, mb,
                           preferred_element_type=jnp.float32)
            for k in range(NB):
                row_s[k:k + 1, :] = rowc[0:1, k * RB:(k + 1) * RB]

    @pl.when(i == NB)
    def _agg_phase():
        A = jnp.zeros((8, D), jnp.float32)
        sc = jnp.zeros((8, 1), jnp.float32)
        S = jnp.float32(0.0)
        eye = (lax.broadcasted_iota(jnp.int32, (RB, RB), 0)
               == lax.broadcasted_iota(jnp.int32, (RB, RB), 1)
               ).astype(jnp.float32)
        for k in range(NB):
            rowk = row_s[k:k + 1, :]
            nbk = rowk != 0
            dck = deg_s[k * RB:(k + 1) * RB, 0:1]
            degk = lax.dot_general(dck, eye, (((0,), (0,)), ((), ())),
                                   preferred_element_type=jnp.float32)
            wtk = jnp.where(
                nbk, lax.rsqrt(jnp.where(nbk, degk, 1.0)), 0.0)
            lwTk = lmT_ref[:, k * RB:(k + 1) * RB] * wtk
            xk = x_ref[k * RB:(k + 1) * RB, :]
            A = A + jnp.dot(lwTk, xk, preferred_element_type=jnp.float32)
            sc = sc + jnp.sum(lwTk, axis=1, keepdims=True)
            S = S + jnp.sum(rowk)
        rs = jnp.where(S > 0, lax.rsqrt(S), 0.0)
        sel = (lax.broadcasted_iota(jnp.int32, (1, N), 1)
               == idx).astype(jnp.float32)
        XI = jnp.dot(sel, x_ref[...],
                     preferred_element_type=jnp.float32)
        flagv = spref[1]
        Wsel = jnp.where(flagv == 1, W1_ref[...], W2_ref[...])
        bsel = jnp.where(flagv == 1, b1_ref[...], b2_ref[...])
        SB = (sc * rs) * bsel
        ta = jnp.maximum(
            jnp.dot(A * rs, Wsel, preferred_element_type=jnp.float32) + SB,
            0.0)
        zi = jnp.maximum(
            jnp.dot(XI, Wsel, preferred_element_type=jnp.float32) + bsel,
            0.0)
        h = jnp.concatenate(
            [zi] + [ta[l:l + 1, :] for l in range(7)], axis=1)
        P = jnp.maximum(XI, h)
        out_ref[...] = (jnp.dot(P, Wp_ref[...],
                                preferred_element_type=jnp.float32)
                        + bp_ref[...])


def kernel(flag, index, matrix, x_features, x_labels, W1, b1, W2, b2, Wp, bp):
    spref = jnp.array([index, flag]).astype(jnp.int32)
    lmT = (x_labels != 0).astype(jnp.float32).T
    lmT8 = jnp.concatenate(
        [lmT, jnp.zeros((1, N), jnp.float32)], axis=0)
    grid_spec = pltpu.PrefetchScalarGridSpec(
        num_scalar_prefetch=1,
        grid=(NB + 1,),
        in_specs=[
            pl.BlockSpec((RB, N), lambda i, s: (jnp.minimum(i, NB - 1), 0)),
            pl.BlockSpec((N, D), lambda i, s: (0, 0)),
            pl.BlockSpec((8, N), lambda i, s: (0, 0)),
            pl.BlockSpec((D, 64), lambda i, s: (0, 0)),
            pl.BlockSpec((1, 64), lambda i, s: (0, 0)),
            pl.BlockSpec((D, 64), lambda i, s: (0, 0)),
            pl.BlockSpec((1, 64), lambda i, s: (0, 0)),
            pl.BlockSpec((D, 7), lambda i, s: (0, 0)),
            pl.BlockSpec((1, 7), lambda i, s: (0, 0)),
        ],
        out_specs=pl.BlockSpec((1, 7), lambda i, s: (0, 0)),
        scratch_shapes=[
            pltpu.VMEM((N, 8), jnp.float32),
            pltpu.VMEM((NB, RB), jnp.float32),
        ],
    )
    return pl.pallas_call(
        _body,
        grid_spec=grid_spec,
        out_shape=jax.ShapeDtypeStruct((1, 7), jnp.float32),
    )(spref, matrix, x_features, lmT8,
      W1, b1.reshape(1, 64), W2, b2.reshape(1, 64), Wp, bp.reshape(1, 7))


# speedup vs baseline: 1.1795x; 1.1795x over previous
import jax
import jax.numpy as jnp
from jax import lax
from jax.experimental import pallas as pl
from jax.experimental.pallas import tpu as pltpu

N = 4096
D = 512
RB = 512
NB = N // RB


def _deg_body(spref, m_ref, deg_ref, row_ref):
    i = pl.program_id(0)
    mb = m_ref[...]
    ones = jnp.ones((1, N), dtype=jnp.float32)
    deg_ref[...] = lax.dot_general(
        ones, mb, (((1,), (1,)), ((), ())),
        preferred_element_type=jnp.float32)

    @pl.when(i == 0)
    def _init():
        row_ref[...] = jnp.zeros_like(row_ref)

    rel = spref[0] - i * RB

    @pl.when((rel >= 0) & (rel < RB))
    def _extract_row():
        sel = (lax.broadcasted_iota(jnp.int32, (1, RB), 1)
               == rel).astype(jnp.float32)
        row_ref[...] += jnp.dot(sel, mb, preferred_element_type=jnp.float32)


def _deg_tc(spref, matrix):
    grid_spec = pltpu.PrefetchScalarGridSpec(
        num_scalar_prefetch=1,
        grid=(NB,),
        in_specs=[pl.BlockSpec((RB, N), lambda i, s: (i, 0))],
        out_specs=[
            pl.BlockSpec((1, RB), lambda i, s: (0, i)),
            pl.BlockSpec((1, N), lambda i, s: (0, 0)),
        ],
    )
    return pl.pallas_call(
        _deg_body,
        grid_spec=grid_spec,
        out_shape=[jax.ShapeDtypeStruct((1, N), jnp.float32),
                   jax.ShapeDtypeStruct((1, N), jnp.float32)],
    )(spref, matrix)


def _main_body(spref, deg_ref, row_ref, x_ref, lmT_ref,
               W1_ref, b1_ref, W2_ref, b2_ref, Wp_ref, bp_ref, out_ref):
    row = row_ref[...]
    nb = row != 0
    wt = jnp.where(nb, lax.rsqrt(jnp.where(nb, deg_ref[...], 1.0)), 0.0)
    lwT = lmT_ref[...] * wt
    xw = x_ref[...]
    A = jnp.dot(lwT, xw, preferred_element_type=jnp.float32)
    sc = jnp.sum(lwT, axis=1, keepdims=True)
    S = jnp.sum(row)
    rs = jnp.where(S > 0, lax.rsqrt(S), 0.0)
    idx = spref[0]
    sel = (lax.broadcasted_iota(jnp.int32, (1, N), 1)
           == idx).astype(jnp.float32)
    XI = jnp.dot(sel, xw, preferred_element_type=jnp.float32)
    flagv = spref[1]
    Wsel = jnp.where(flagv == 1, W1_ref[...], W2_ref[...])
    bsel = jnp.where(flagv == 1, b1_ref[...], b2_ref[...])
    SB = (sc * rs) * bsel
    ta = jnp.maximum(
        jnp.dot(A * rs, Wsel, preferred_element_type=jnp.float32) + SB, 0.0)
    zi = jnp.maximum(
        jnp.dot(XI, Wsel, preferred_element_type=jnp.float32) + bsel, 0.0)
    h = jnp.concatenate(
        [zi] + [ta[l:l + 1, :] for l in range(7)], axis=1)
    P = jnp.maximum(XI, h)
    out_ref[...] = (jnp.dot(P, Wp_ref[...],
                            preferred_element_type=jnp.float32)
                    + bp_ref[...])


def _main_tc(spref, deg_row, mrow, x, lmT8, W1, b1, W2, b2, Wp, bp):
    grid_spec = pltpu.PrefetchScalarGridSpec(
        num_scalar_prefetch=1,
        grid=(1,),
        in_specs=[
            pl.BlockSpec((1, N), lambda i, s: (0, 0)),
            pl.BlockSpec((1, N), lambda i, s: (0, 0)),
            pl.BlockSpec((N, D), lambda i, s: (0, 0)),
            pl.BlockSpec((8, N), lambda i, s: (0, 0)),
            pl.BlockSpec((D, 64), lambda i, s: (0, 0)),
            pl.BlockSpec((1, 64), lambda i, s: (0, 0)),
            pl.BlockSpec((D, 64), lambda i, s: (0, 0)),
            pl.BlockSpec((1, 64), lambda i, s: (0, 0)),
            pl.BlockSpec((D, 7), lambda i, s: (0, 0)),
            pl.BlockSpec((1, 7), lambda i, s: (0, 0)),
        ],
        out_specs=pl.BlockSpec((1, 7), lambda i, s: (0, 0)),
    )
    return pl.pallas_call(
        _main_body,
        grid_spec=grid_spec,
        out_shape=jax.ShapeDtypeStruct((1, 7), jnp.float32),
    )(spref, deg_row, mrow, x, lmT8, W1, b1, W2, b2, Wp, bp)


def kernel(flag, index, matrix, x_features, x_labels, W1, b1, W2, b2, Wp, bp):
    spref = jnp.array([index, flag]).astype(jnp.int32)
    deg_row, mrow = _deg_tc(spref, matrix)
    lmT = (x_labels != 0).astype(jnp.float32).T
    lmT8 = jnp.concatenate(
        [lmT, jnp.zeros((1, N), jnp.float32)], axis=0)
    return _main_tc(spref, deg_row, mrow, x_features, lmT8,
                    W1, b1.reshape(1, 64), W2, b2.reshape(1, 64),
                    Wp, bp.reshape(1, 7))
